# baseline (speedup 1.0000x reference)
import functools

import jax
import jax.numpy as jnp
from jax import lax
from jax.experimental import pallas as pl
from jax.experimental.pallas import tpu as pltpu
from jax.experimental.pallas import tpu_sc as plsc

_P = 16


def kernel(x, pe_weight):
    B, S, D = x.shape
    mesh = plsc.VectorSubcoreMesh(core_axis_name="c", subcore_axis_name="s")
    nw = mesh.num_cores * mesh.num_subcores
    rows_per_w = S // nw
    nchunks = rows_per_w // _P
    dchunks = D // 16

    @functools.partial(
        pl.kernel,
        out_type=jax.ShapeDtypeStruct((B, S, D), jnp.float32),
        mesh=mesh,
        scratch_types=[
            pltpu.VMEM((_P, D), jnp.float32),
            pltpu.VMEM((_P, D), jnp.float32),
        ],
    )
    def run(x_hbm, pe_hbm, out_hbm, pe_v, x_v):
        wid = lax.axis_index("s") * mesh.num_cores + lax.axis_index("c")
        base = wid * rows_per_w

        def chunk_body(c, _):
            seq0 = base + c * _P
            pltpu.sync_copy(pe_hbm.at[pl.ds(seq0, _P)], pe_v)

            def batch_body(b, _):
                pltpu.sync_copy(x_hbm.at[b, pl.ds(seq0, _P)], x_v)

                def row_body(i, _):
                    def col_body(j, _):
                        plsc.addupdate(
                            x_v.at[i, pl.ds(j * 16, 16)],
                            pe_v[i, pl.ds(j * 16, 16)],
                        )
                        return 0

                    return lax.fori_loop(0, dchunks, col_body, 0)

                lax.fori_loop(0, _P, row_body, 0)
                pltpu.sync_copy(x_v, out_hbm.at[b, pl.ds(seq0, _P)])
                return 0

            lax.fori_loop(0, B, batch_body, 0)
            return 0

        lax.fori_loop(0, nchunks, chunk_body, 0)

    return run(x, pe_weight)


# speedup vs baseline: 1.4860x; 1.4860x over previous
import functools

import jax
import jax.numpy as jnp
from jax import lax
from jax.experimental import pallas as pl
from jax.experimental.pallas import tpu as pltpu
from jax.experimental.pallas import tpu_sc as plsc

_P = 16


def kernel(x, pe_weight):
    B, S, D = x.shape
    mesh = plsc.VectorSubcoreMesh(core_axis_name="c", subcore_axis_name="s")
    nw = mesh.num_cores * mesh.num_subcores
    rows_per_w = S // nw
    nchunks = rows_per_w // _P
    dchunks = D // 16

    n_xbuf = 2 * B
    scratch = (
        [pltpu.VMEM((_P, D), jnp.float32) for _ in range(n_xbuf)]
        + [pltpu.VMEM((_P, D), jnp.float32) for _ in range(2)]
        + [pltpu.SemaphoreType.DMA for _ in range(n_xbuf)]
        + [pltpu.SemaphoreType.DMA for _ in range(n_xbuf)]
        + [pltpu.SemaphoreType.DMA for _ in range(2)]
    )

    @functools.partial(
        pl.kernel,
        out_type=jax.ShapeDtypeStruct((B, S, D), jnp.float32),
        mesh=mesh,
        scratch_types=scratch,
    )
    def run(x_hbm, pe_hbm, out_hbm, *bufs):
        xbuf = bufs[:n_xbuf]
        pebuf = bufs[n_xbuf : n_xbuf + 2]
        in_sem = bufs[n_xbuf + 2 : 2 * n_xbuf + 2]
        out_sem = bufs[2 * n_xbuf + 2 : 3 * n_xbuf + 2]
        pe_sem = bufs[3 * n_xbuf + 2 :]

        wid = lax.axis_index("s") * mesh.num_cores + lax.axis_index("c")
        base = wid * rows_per_w

        def seq0(c):
            return base + c * _P

        def pe_copy(c):
            return pltpu.make_async_copy(
                pe_hbm.at[pl.ds(seq0(c), _P)], pebuf[c % 2], pe_sem[c % 2]
            )

        def in_copy(c, b):
            slot = (c % 2) * B + b
            return pltpu.make_async_copy(
                x_hbm.at[b, pl.ds(seq0(c), _P)], xbuf[slot], in_sem[slot]
            )

        def out_copy(c, b):
            slot = (c % 2) * B + b
            return pltpu.make_async_copy(
                xbuf[slot], out_hbm.at[b, pl.ds(seq0(c), _P)], out_sem[slot]
            )

        pe_copy(0).start()
        for b in range(B):
            in_copy(0, b).start()

        for c in range(nchunks):
            pe_copy(c).wait()
            if c + 1 < nchunks:
                pe_copy(c + 1).start()
            pe_v = pebuf[c % 2]
            for b in range(B):
                slot = (c % 2) * B + b
                in_copy(c, b).wait()
                x_v = xbuf[slot]

                def row_body(i, _):
                    def col_body(j, _):
                        plsc.addupdate(
                            x_v.at[i, pl.ds(j * 16, 16)],
                            pe_v[i, pl.ds(j * 16, 16)],
                        )
                        return 0

                    return lax.fori_loop(0, dchunks, col_body, 0)

                lax.fori_loop(0, _P, row_body, 0)
                out_copy(c, b).start()
                if c + 1 < nchunks:
                    if c >= 1:
                        out_copy(c - 1, b).wait()
                    in_copy(c + 1, b).start()

        for b in range(B):
            out_copy(nchunks - 2, b).wait()
            out_copy(nchunks - 1, b).wait()

    return run(x, pe_weight)
